# baseline (speedup 1.0000x reference)
import jax
import jax.numpy as jnp
from jax import lax
from jax.experimental import pallas as pl
from jax.experimental.pallas import tpu as pltpu
from jax.experimental.pallas import tpu_sc as plsc

NUM_ENT = 1000000
NUM_REL = 1000
EMB_DIM = 64
BATCH = 16384
GAMMA = 2.0

NC = 2
NS = 16
L = 16
NW = NC * NS
B_PER_W = BATCH // NW
CHUNK = 128
NCHUNK = B_PER_W // CHUNK
GROUPS = B_PER_W // L


def _vsqrt(x):
    y = jnp.where(x < 0.0125, jnp.float32(0.05),
        jnp.where(x < 0.125, jnp.float32(0.2),
        jnp.where(x < 0.7, jnp.float32(0.54), jnp.float32(1.12))))
    for _ in range(5):
        y = 0.5 * (y + x / y)
    return jnp.where(x < 1e-12, jnp.float32(0.0), y)


def _body(hs_hbm, rs_hbm, ts_hbm, entT_hbm, relT_hbm, out_hbm,
          idx_h, idx_r, idx_t, h_buf, r_buf, t_buf, out_v, sem):
    wid = lax.axis_index("s") * NC + lax.axis_index("c")
    base = wid * B_PER_W

    pltpu.sync_copy(hs_hbm.at[pl.ds(base, B_PER_W)], idx_h)
    pltpu.sync_copy(rs_hbm.at[pl.ds(base, B_PER_W)], idx_r)
    pltpu.sync_copy(ts_hbm.at[pl.ds(base, B_PER_W)], idx_t)

    def issue(d, carry):
        for c in range(NCHUNK):
            sl = pl.ds(c * CHUNK, CHUNK)
            pltpu.async_copy(
                entT_hbm.at[d].at[idx_h.at[sl]], h_buf.at[d, sl], sem)
            pltpu.async_copy(
                relT_hbm.at[d].at[idx_r.at[sl]], r_buf.at[d, sl], sem)
            pltpu.async_copy(
                entT_hbm.at[d].at[idx_t.at[sl]], t_buf.at[d, sl], sem)
        return carry

    lax.fori_loop(0, EMB_DIM, issue, 0)

    def drain(d, carry):
        pltpu.make_async_copy(
            entT_hbm.at[0, pl.ds(0, B_PER_W)], h_buf.at[d], sem).wait()
        pltpu.make_async_copy(
            entT_hbm.at[0, pl.ds(0, B_PER_W)], r_buf.at[d], sem).wait()
        pltpu.make_async_copy(
            entT_hbm.at[0, pl.ds(0, B_PER_W)], t_buf.at[d], sem).wait()
        return carry

    lax.fori_loop(0, EMB_DIM, drain, 0)

    def group(g, carry):
        sl = pl.ds(g * L, L)
        acc = jnp.zeros((L,), jnp.float32)
        for d in range(EMB_DIM):
            dv = h_buf[d, sl] + r_buf[d, sl] - t_buf[d, sl]
            acc = acc + dv * dv
        out_v[sl] = GAMMA - _vsqrt(acc)
        return carry

    lax.fori_loop(0, GROUPS, group, 0)

    pltpu.sync_copy(out_v, out_hbm.at[pl.ds(base, B_PER_W)])


@jax.jit
def _transe(hs, rs, ts, ent_embs, rel_embs):
    mesh = plsc.VectorSubcoreMesh(
        core_axis_name="c", subcore_axis_name="s",
        num_cores=NC, num_subcores=NS)
    run = pl.kernel(
        _body,
        out_type=jax.ShapeDtypeStruct((BATCH,), jnp.float32),
        mesh=mesh,
        compiler_params=pltpu.CompilerParams(use_tc_tiling_on_sc=False),
        scratch_types=[
            pltpu.VMEM((B_PER_W,), jnp.int32),
            pltpu.VMEM((B_PER_W,), jnp.int32),
            pltpu.VMEM((B_PER_W,), jnp.int32),
            pltpu.VMEM((EMB_DIM, B_PER_W), jnp.float32),
            pltpu.VMEM((EMB_DIM, B_PER_W), jnp.float32),
            pltpu.VMEM((EMB_DIM, B_PER_W), jnp.float32),
            pltpu.VMEM((B_PER_W,), jnp.float32),
            pltpu.SemaphoreType.DMA,
        ],
    )
    return run(hs, rs, ts, ent_embs.T, rel_embs.T)


def kernel(hs, rs, ts, ent_embs, rel_embs):
    score = _transe(hs.astype(jnp.int32), rs.astype(jnp.int32),
                    ts.astype(jnp.int32), ent_embs, rel_embs)
    return score.reshape(-1, 1)


# speedup vs baseline: 8.1808x; 8.1808x over previous
import jax
import jax.numpy as jnp
from jax import lax
from jax.experimental import pallas as pl
from jax.experimental.pallas import tpu as pltpu
from jax.experimental.pallas import tpu_sc as plsc

NUM_ENT = 1000000
NUM_REL = 1000
EMB_DIM = 64
BATCH = 16384
GAMMA = 2.0

NC = 2
NS = 16
L = 16
NW = NC * NS
B_PER_W = BATCH // NW
HALF = B_PER_W // 2
CHUNK = 128
NCHUNK = HALF // CHUNK
HGROUPS = HALF // L
GROUPS = B_PER_W // L
DREG = EMB_DIM // L
PAIR = 2 * EMB_DIM

_PERM_DNUMS = lax.GatherDimensionNumbers(
    offset_dims=(), collapsed_slice_dims=(0,), start_index_map=(0,))


def _lane_perm(x, idx):
    return lax.gather(x, idx[:, None], _PERM_DNUMS, slice_sizes=(1,),
                      mode=lax.GatherScatterMode.PROMISE_IN_BOUNDS)


def _vsqrt(x):
    y = jnp.where(x < 0.0125, jnp.float32(0.05),
        jnp.where(x < 0.125, jnp.float32(0.2),
        jnp.where(x < 0.7, jnp.float32(0.54), jnp.float32(1.12))))
    for _ in range(5):
        y = 0.5 * (y + x / y)
    return jnp.where(x < 1e-12, jnp.float32(0.0), y)


def _body(hs_hbm, rs_hbm, ts_hbm, ent2_hbm, rel2_hbm, out_hbm,
          idx_h, idx_r, idx_t, id2_h, id2_r, id2_t,
          h_buf, r_buf, t_buf, out_v, sem):
    wid = lax.axis_index("s") * NC + lax.axis_index("c")
    base = wid * B_PER_W

    pltpu.sync_copy(hs_hbm.at[pl.ds(base, B_PER_W)], idx_h)
    pltpu.sync_copy(rs_hbm.at[pl.ds(base, B_PER_W)], idx_r)
    pltpu.sync_copy(ts_hbm.at[pl.ds(base, B_PER_W)], idx_t)

    def halve(g, carry):
        sl = pl.ds(g * L, L)
        id2_h[sl] = idx_h[sl] >> 1
        id2_r[sl] = idx_r[sl] >> 1
        id2_t[sl] = idx_t[sl] >> 1
        return carry

    lax.fori_loop(0, GROUPS, halve, 0)

    lane = lax.iota(jnp.int32, L)

    for p in range(2):
        off = p * HALF

        cps = []
        for cch in range(NCHUNK):
            src = pl.ds(off + cch * CHUNK, CHUNK)
            dst = pl.ds(cch * CHUNK, CHUNK)
            cps.append(pltpu.async_copy(
                ent2_hbm.at[id2_h.at[src]], h_buf.at[dst], sem))
            cps.append(pltpu.async_copy(
                rel2_hbm.at[id2_r.at[src]], r_buf.at[dst], sem))
            cps.append(pltpu.async_copy(
                ent2_hbm.at[id2_t.at[src]], t_buf.at[dst], sem))
        for cp in cps:
            cp.wait()

        def group(g, carry):
            gsl = pl.ds(off + g * L, L)
            ph = (idx_h[gsl] & 1) * EMB_DIM
            pr = (idx_r[gsl] & 1) * EMB_DIM
            pt = (idx_t[gsl] & 1) * EMB_DIM
            sums = jnp.zeros((L,), jnp.float32)
            for j in range(L):
                i = g * L + j
                bh = ph[j]
                br = pr[j]
                bt = pt[j]
                acc = jnp.zeros((L,), jnp.float32)
                for c in range(DREG):
                    hv = h_buf[i, pl.ds(bh + c * L, L)]
                    rv = r_buf[i, pl.ds(br + c * L, L)]
                    tv = t_buf[i, pl.ds(bt + c * L, L)]
                    d = hv + rv - tv
                    acc = acc + d * d
                for k in (8, 4, 2, 1):
                    acc = acc + _lane_perm(acc, lane ^ k)
                sums = jnp.where(lane == j, acc, sums)
            out_v[pl.ds(off + g * L, L)] = GAMMA - _vsqrt(sums)
            return carry

        lax.fori_loop(0, HGROUPS, group, 0)

    pltpu.sync_copy(out_v, out_hbm.at[pl.ds(base, B_PER_W)])


@jax.jit
def _transe(hs, rs, ts, ent_embs, rel_embs):
    mesh = plsc.VectorSubcoreMesh(
        core_axis_name="c", subcore_axis_name="s",
        num_cores=NC, num_subcores=NS)
    run = pl.kernel(
        _body,
        out_type=jax.ShapeDtypeStruct((BATCH,), jnp.float32),
        mesh=mesh,
        scratch_types=[
            pltpu.VMEM((B_PER_W,), jnp.int32),
            pltpu.VMEM((B_PER_W,), jnp.int32),
            pltpu.VMEM((B_PER_W,), jnp.int32),
            pltpu.VMEM((B_PER_W,), jnp.int32),
            pltpu.VMEM((B_PER_W,), jnp.int32),
            pltpu.VMEM((B_PER_W,), jnp.int32),
            pltpu.VMEM((HALF, PAIR), jnp.float32),
            pltpu.VMEM((HALF, PAIR), jnp.float32),
            pltpu.VMEM((HALF, PAIR), jnp.float32),
            pltpu.VMEM((B_PER_W,), jnp.float32),
            pltpu.SemaphoreType.DMA,
        ],
    )
    ent2 = ent_embs.reshape(NUM_ENT // 2, PAIR)
    rel2 = rel_embs.reshape(NUM_REL // 2, PAIR)
    return run(hs, rs, ts, ent2, rel2)


def kernel(hs, rs, ts, ent_embs, rel_embs):
    score = _transe(hs.astype(jnp.int32), rs.astype(jnp.int32),
                    ts.astype(jnp.int32), ent_embs, rel_embs)
    return score.reshape(-1, 1)


# speedup vs baseline: 11.6544x; 1.4246x over previous
import jax
import jax.numpy as jnp
from jax import lax
from jax.experimental import pallas as pl
from jax.experimental.pallas import tpu as pltpu
from jax.experimental.pallas import tpu_sc as plsc

NUM_ENT = 1000000
NUM_REL = 1000
EMB_DIM = 64
BATCH = 16384
GAMMA = 2.0

NC = 2
NS = 16
L = 16
NW = NC * NS
B_PER_W = BATCH // NW
CHUNK = 32
NCHUNK = B_PER_W // CHUNK
CGROUPS = CHUNK // L
ROWG = 8

_PERM_DNUMS = lax.GatherDimensionNumbers(
    offset_dims=(), collapsed_slice_dims=(0,), start_index_map=(0,))


def _lane_perm(x, idx):
    return lax.gather(x, idx[:, None], _PERM_DNUMS, slice_sizes=(1,),
                      mode=lax.GatherScatterMode.PROMISE_IN_BOUNDS)


def _vsqrt(x):
    y = jnp.where(x < 0.0125, jnp.float32(0.05),
        jnp.where(x < 0.125, jnp.float32(0.2),
        jnp.where(x < 0.7, jnp.float32(0.54), jnp.float32(1.12))))
    for _ in range(5):
        y = 0.5 * (y + x / y)
    return jnp.where(x < 1e-12, jnp.float32(0.0), y)


def _body(hs_hbm, rs_hbm, ts_hbm, ent_hbm, rel_hbm, out_hbm,
          idx_h, idx_r, idx_t, h_buf, r_buf, t_buf, out_v, sem):
    wid = lax.axis_index("s") * NC + lax.axis_index("c")
    base = wid * B_PER_W

    pltpu.sync_copy(hs_hbm.at[pl.ds(base, B_PER_W)], idx_h)
    pltpu.sync_copy(rs_hbm.at[pl.ds(base, B_PER_W)], idx_r)
    pltpu.sync_copy(ts_hbm.at[pl.ds(base, B_PER_W)], idx_t)

    lane = lax.iota(jnp.int32, L)

    def chunk(ch, carry):
        coff = ch * CHUNK
        cps = []
        ivs_h, ivs_r, ivs_t = [], [], []
        for g in range(CGROUPS):
            gsl = pl.ds(coff + g * L, L)
            ivh = idx_h[gsl]
            ivr = idx_r[gsl]
            ivt = idx_t[gsl]
            ivs_h.append(ivh)
            ivs_r.append(ivr)
            ivs_t.append(ivt)
            for j in range(L):
                slot = g * L + j
                bh = pl.multiple_of((ivh[j] >> 3) * ROWG, ROWG)
                br = pl.multiple_of((ivr[j] >> 3) * ROWG, ROWG)
                bt = pl.multiple_of((ivt[j] >> 3) * ROWG, ROWG)
                cps.append(pltpu.async_copy(
                    ent_hbm.at[pl.ds(bh, ROWG), :], h_buf.at[slot], sem))
                cps.append(pltpu.async_copy(
                    rel_hbm.at[pl.ds(br, ROWG), :], r_buf.at[slot], sem))
                cps.append(pltpu.async_copy(
                    ent_hbm.at[pl.ds(bt, ROWG), :], t_buf.at[slot], sem))
        for cp in cps:
            cp.wait()

        for g in range(CGROUPS):
            mh = ivs_h[g] & 7
            mr = ivs_r[g] & 7
            mt = ivs_t[g] & 7
            sums = jnp.zeros((L,), jnp.float32)
            for j in range(L):
                slot = g * L + j
                acc = jnp.zeros((L,), jnp.float32)
                for c in range(EMB_DIM // L):
                    sl = pl.ds(c * L, L)
                    d = (h_buf[slot, mh[j], sl] + r_buf[slot, mr[j], sl]
                         - t_buf[slot, mt[j], sl])
                    acc = acc + d * d
                for k in (8, 4, 2, 1):
                    acc = acc + _lane_perm(acc, lane ^ k)
                sums = jnp.where(lane == j, acc, sums)
            out_v[pl.ds(coff + g * L, L)] = GAMMA - _vsqrt(sums)
        return carry

    lax.fori_loop(0, NCHUNK, chunk, 0)

    pltpu.sync_copy(out_v, out_hbm.at[pl.ds(base, B_PER_W)])


@jax.jit
def _transe(hs, rs, ts, ent_embs, rel_embs):
    mesh = plsc.VectorSubcoreMesh(
        core_axis_name="c", subcore_axis_name="s",
        num_cores=NC, num_subcores=NS)
    run = pl.kernel(
        _body,
        out_type=jax.ShapeDtypeStruct((BATCH,), jnp.float32),
        mesh=mesh,
        scratch_types=[
            pltpu.VMEM((B_PER_W,), jnp.int32),
            pltpu.VMEM((B_PER_W,), jnp.int32),
            pltpu.VMEM((B_PER_W,), jnp.int32),
            pltpu.VMEM((CHUNK, ROWG, EMB_DIM), jnp.float32),
            pltpu.VMEM((CHUNK, ROWG, EMB_DIM), jnp.float32),
            pltpu.VMEM((CHUNK, ROWG, EMB_DIM), jnp.float32),
            pltpu.VMEM((B_PER_W,), jnp.float32),
            pltpu.SemaphoreType.DMA,
        ],
    )
    return run(hs, rs, ts, ent_embs, rel_embs)


def kernel(hs, rs, ts, ent_embs, rel_embs):
    score = _transe(hs.astype(jnp.int32), rs.astype(jnp.int32),
                    ts.astype(jnp.int32), ent_embs, rel_embs)
    return score.reshape(-1, 1)


# speedup vs baseline: 14.1381x; 1.2131x over previous
import jax
import jax.numpy as jnp
from jax import lax
from jax.experimental import pallas as pl
from jax.experimental.pallas import tpu as pltpu
from jax.experimental.pallas import tpu_sc as plsc

NUM_ENT = 1000000
NUM_REL = 1000
EMB_DIM = 64
BATCH = 16384
GAMMA = 2.0

NC = 2
NS = 16
L = 16
NW = NC * NS
B_PER_W = BATCH // NW
CHUNK = 128
NCHUNK = B_PER_W // CHUNK
GROUPS = B_PER_W // L
DREG = EMB_DIM // L

_PERM_DNUMS = lax.GatherDimensionNumbers(
    offset_dims=(), collapsed_slice_dims=(0,), start_index_map=(0,))


def _lane_perm(x, idx):
    return lax.gather(x, idx[:, None], _PERM_DNUMS, slice_sizes=(1,),
                      mode=lax.GatherScatterMode.PROMISE_IN_BOUNDS)


def _vsqrt(x):
    y = jnp.where(x < 0.0125, jnp.float32(0.05),
        jnp.where(x < 0.125, jnp.float32(0.2),
        jnp.where(x < 0.7, jnp.float32(0.54), jnp.float32(1.12))))
    for _ in range(5):
        y = 0.5 * (y + x / y)
    return jnp.where(x < 1e-12, jnp.float32(0.0), y)


def _body(hs_hbm, rs_hbm, ts_hbm, ent_hbm, rel_hbm, out_hbm,
          idx_h, idx_r, idx_t, h_buf, r_buf, t_buf, out_v, sem):
    wid = lax.axis_index("s") * NC + lax.axis_index("c")
    base = wid * B_PER_W

    pltpu.sync_copy(hs_hbm.at[pl.ds(base, B_PER_W)], idx_h)
    pltpu.sync_copy(rs_hbm.at[pl.ds(base, B_PER_W)], idx_r)
    pltpu.sync_copy(ts_hbm.at[pl.ds(base, B_PER_W)], idx_t)

    cps = []
    for j in range(NCHUNK):
        rows = pl.ds(j * CHUNK, CHUNK)
        cps.append(pltpu.async_copy(
            ent_hbm.at[idx_h.at[rows]], h_buf.at[rows], sem))
        cps.append(pltpu.async_copy(
            rel_hbm.at[idx_r.at[rows]], r_buf.at[rows], sem))
        cps.append(pltpu.async_copy(
            ent_hbm.at[idx_t.at[rows]], t_buf.at[rows], sem))
    for cp in cps:
        cp.wait()

    lane = lax.iota(jnp.int32, L)

    def group(g, carry):
        sums = jnp.zeros((L,), jnp.float32)
        for j in range(L):
            i = g * L + j
            acc = jnp.zeros((L,), jnp.float32)
            for c in range(DREG):
                sl = pl.ds(c * L, L)
                d = h_buf[i, sl] + r_buf[i, sl] - t_buf[i, sl]
                acc = acc + d * d
            for k in (8, 4, 2, 1):
                acc = acc + _lane_perm(acc, lane ^ k)
            sums = jnp.where(lane == j, acc, sums)
        out_v[pl.ds(g * L, L)] = GAMMA - _vsqrt(sums)
        return carry

    lax.fori_loop(0, GROUPS, group, 0)

    pltpu.sync_copy(out_v, out_hbm.at[pl.ds(base, B_PER_W)])


@jax.jit
def _transe(hs, rs, ts, ent_embs, rel_embs):
    mesh = plsc.VectorSubcoreMesh(
        core_axis_name="c", subcore_axis_name="s",
        num_cores=NC, num_subcores=NS)
    run = pl.kernel(
        _body,
        out_type=jax.ShapeDtypeStruct((BATCH,), jnp.float32),
        mesh=mesh,
        compiler_params=pltpu.CompilerParams(use_tc_tiling_on_sc=False),
        scratch_types=[
            pltpu.VMEM((B_PER_W,), jnp.int32),
            pltpu.VMEM((B_PER_W,), jnp.int32),
            pltpu.VMEM((B_PER_W,), jnp.int32),
            pltpu.VMEM((B_PER_W, EMB_DIM), jnp.float32),
            pltpu.VMEM((B_PER_W, EMB_DIM), jnp.float32),
            pltpu.VMEM((B_PER_W, EMB_DIM), jnp.float32),
            pltpu.VMEM((B_PER_W,), jnp.float32),
            pltpu.SemaphoreType.DMA,
        ],
    )
    from jax._src.pjit import with_layout_constraint
    from jax._src.layout import Layout as _Layout
    cl = _Layout(major_to_minor=(0, 1), tiling=())
    entc = with_layout_constraint(ent_embs, cl)
    relc = with_layout_constraint(rel_embs, cl)
    return run(hs, rs, ts, entc, relc)


def kernel(hs, rs, ts, ent_embs, rel_embs):
    score = _transe(hs.astype(jnp.int32), rs.astype(jnp.int32),
                    ts.astype(jnp.int32), ent_embs, rel_embs)
    return score.reshape(-1, 1)
